# Initial kernel scaffold; baseline (speedup 1.0000x reference)
import jax
import jax.numpy as jnp
from jax.experimental import pallas as pl


def kernel(x, edge_index, edge_attr, edge_weight, eps, W_be, b_be, W1, b1, W2, b2):
    raise NotImplementedError("write your pallas kernel here")


# speedup vs baseline: 2.1547x; 2.1547x over previous
import functools

import jax
import jax.numpy as jnp
from jax import lax
from jax.experimental import pallas as pl
from jax.experimental.pallas import tpu as pltpu
from jax.experimental.pallas import tpu_sc as plsc

NC = 2
NS = 16
NW = NC * NS
CH = 128


def _gelu_exact(v):
    z = jnp.abs(v) * 0.7071067811865476
    t = 1.0 / (1.0 + 0.3275911 * z)
    poly = ((((1.061405429 * t - 1.453152027) * t + 1.421413741) * t
             - 0.284496736) * t + 0.254829592) * t
    erf_abs = 1.0 - poly * jnp.exp(-z * z)
    erf = jnp.sign(v) * erf_abs
    return 0.5 * v * (1.0 + erf)


def _gather_body(kw, x_hbm, srcg_hbm, out_hbm, idx_v, row_v, sem):
    c = lax.axis_index("c")
    s = lax.axis_index("s")
    wid = s * NC + c
    pltpu.sync_copy(srcg_hbm.at[wid], idx_v)

    def step(j, carry):
        pltpu.async_copy(x_hbm.at[idx_v.at[j]], row_v, sem).wait()
        pltpu.sync_copy(row_v, out_hbm.at[pl.ds((wid * kw + j) * CH, CH)])
        return carry

    lax.fori_loop(0, kw, step, 0)


def _sc_gather(xf, src_g, kw, e_pad, d):
    mesh = plsc.VectorSubcoreMesh(core_axis_name="c", subcore_axis_name="s")
    return pl.kernel(
        functools.partial(_gather_body, kw),
        out_type=jax.ShapeDtypeStruct((e_pad, d), jnp.float32),
        mesh=mesh,
        scratch_types=[
            pltpu.VMEM((kw, CH), jnp.int32),
            pltpu.VMEM((CH, d), jnp.float32),
            pltpu.SemaphoreType.DMA,
        ],
    )(xf, src_g)


def _msg_body(g_ref, attr_ref, w_ref, wbe_ref, bbe_ref, out_ref):
    emb = jnp.dot(attr_ref[...], wbe_ref[...],
                  preferred_element_type=jnp.float32) + bbe_ref[...]
    out_ref[...] = _gelu_exact(g_ref[...] + emb) * w_ref[...]


def _tc_message(g, attr_p, w_p, wbe, bbe, e_pad, d, de, be):
    grid = e_pad // be
    return pl.pallas_call(
        _msg_body,
        grid=(grid,),
        in_specs=[
            pl.BlockSpec((be, d), lambda i: (i, 0)),
            pl.BlockSpec((be, de), lambda i: (i, 0)),
            pl.BlockSpec((be, 1), lambda i: (i, 0)),
            pl.BlockSpec((de, d), lambda i: (0, 0)),
            pl.BlockSpec((1, d), lambda i: (0, 0)),
        ],
        out_specs=pl.BlockSpec((be, d), lambda i: (i, 0)),
        out_shape=jax.ShapeDtypeStruct((e_pad, d), jnp.float32),
    )(g, attr_p, w_p, wbe, bbe)


def _scatter_body(kw, n_pad, msg_hbm, dstg_hbm, out_hbm, idx_v, msg_v, zero_v, acc):
    c = lax.axis_index("c")
    s = lax.axis_index("s")
    wid = s * NC + c
    rows_per_sub = n_pad // NS

    d = msg_v.shape[1]

    def zstep(i, carry):
        def zcol(k2, carry2):
            zero_v[i, pl.ds(k2 * 16, 16)] = jnp.zeros((16,), jnp.float32)
            return carry2

        return lax.fori_loop(0, d // 16, zcol, carry)

    lax.fori_loop(0, CH, zstep, 0)
    base = s * rows_per_sub
    nfull = rows_per_sub // CH
    rem = rows_per_sub - nfull * CH

    def zcopy(i, carry):
        pltpu.sync_copy(zero_v, acc.at[pl.ds(base + i * CH, CH)])
        return carry

    lax.fori_loop(0, nfull, zcopy, 0)
    if rem:
        pltpu.sync_copy(zero_v.at[pl.ds(0, rem)],
                        acc.at[pl.ds(base + nfull * CH, rem)])
    plsc.subcore_barrier()

    pltpu.sync_copy(dstg_hbm.at[wid], idx_v)

    def step(j, carry):
        pltpu.sync_copy(msg_hbm.at[pl.ds((wid * kw + j) * CH, CH)], msg_v)
        pltpu.sync_copy(msg_v, acc.at[idx_v.at[j]], add=True)
        return carry

    lax.fori_loop(0, kw, step, 0)
    plsc.subcore_barrier()
    pltpu.sync_copy(acc.at[pl.ds(base, rows_per_sub)],
                    out_hbm.at[c].at[pl.ds(base, rows_per_sub)])


def _sc_scatter(msg, dst_g, kw, n_pad, d):
    mesh = plsc.VectorSubcoreMesh(core_axis_name="c", subcore_axis_name="s")
    return pl.kernel(
        functools.partial(_scatter_body, kw, n_pad),
        out_type=jax.ShapeDtypeStruct((NC, n_pad, d), jnp.float32),
        mesh=mesh,
        scratch_types=[
            pltpu.VMEM((kw, CH), jnp.int32),
            pltpu.VMEM((CH, d), jnp.float32),
            pltpu.VMEM((CH, d), jnp.float32),
            pltpu.VMEM_SHARED((n_pad, d), jnp.float32),
        ],
    )(msg, dst_g)


def _mlp_body(scale_ref, x_ref, p_ref, w1_ref, b1_ref, w2_ref, b2_ref, out_ref):
    h = scale_ref[0, 0] * x_ref[...] + p_ref[0] + p_ref[1]
    a = _gelu_exact(jnp.dot(h, w1_ref[...], preferred_element_type=jnp.float32)
                    + b1_ref[...])
    out_ref[...] = jnp.dot(a, w2_ref[...],
                           preferred_element_type=jnp.float32) + b2_ref[...]


def _tc_mlp(scale, xf, parts, w1, b1, w2, b2, n, d, bn):
    grid = n // bn
    return pl.pallas_call(
        _mlp_body,
        grid=(grid,),
        in_specs=[
            pl.BlockSpec(memory_space=pltpu.SMEM),
            pl.BlockSpec((bn, d), lambda i: (i, 0)),
            pl.BlockSpec((NC, bn, d), lambda i: (0, i, 0)),
            pl.BlockSpec((d, d), lambda i: (0, 0)),
            pl.BlockSpec((1, d), lambda i: (0, 0)),
            pl.BlockSpec((d, d), lambda i: (0, 0)),
            pl.BlockSpec((1, d), lambda i: (0, 0)),
        ],
        out_specs=pl.BlockSpec((bn, d), lambda i: (i, 0)),
        out_shape=jax.ShapeDtypeStruct((n, d), jnp.float32),
    )(scale, xf, parts, w1, b1, w2, b2)


def kernel(x, edge_index, edge_attr, edge_weight, eps, W_be, b_be, W1, b1, W2, b2):
    r, cdim, n, d = x.shape
    e = edge_index.shape[1]
    de = edge_attr.shape[1]

    kw = -(-e // (NW * CH))
    e_pad = NW * kw * CH
    pad = e_pad - e

    xf = x.reshape(n, d)
    src = edge_index[0]
    dst = edge_index[1]
    src_g = jnp.pad(src, (0, pad)).reshape(NW, kw, CH)
    dst_g = jnp.pad(dst, (0, pad)).reshape(NW, kw, CH)
    attr_p = jnp.pad(edge_attr, ((0, pad), (0, 0)))
    w_p = jnp.pad(edge_weight.reshape(e, 1), ((0, pad), (0, 0)))

    n_pad = -(-n // (NS * 8)) * (NS * 8)
    g = _sc_gather(xf, src_g, kw, e_pad, d)
    msg = _tc_message(g, attr_p, w_p, W_be, b_be.reshape(1, d), e_pad, d, de, 4096)
    parts = _sc_scatter(msg, dst_g, kw, n_pad, d)
    scale = (1.0 + eps).reshape(1, 1)
    out = _tc_mlp(scale, xf, parts, W1, b1.reshape(1, d), W2, b2.reshape(1, d),
                  n, d, 1000)
    return out.reshape(x.shape)
